# Initial kernel scaffold; baseline (speedup 1.0000x reference)
import jax
import jax.numpy as jnp
from jax.experimental import pallas as pl


def kernel(discrete_actions, continuous_actions, emb_tables, lin_w, lin_b, component_tokens):
    raise NotImplementedError("write your pallas kernel here")


# speedup vs baseline: 1.8742x; 1.8742x over previous
import functools

import jax
import jax.numpy as jnp
from jax import lax
from jax.experimental import pallas as pl
from jax.experimental.pallas import tpu as pltpu
from jax.experimental.pallas import tpu_sc as plsc

_B = 16
_T = 256
_ND = 4
_NC = 6
_BINS = 256
_SA = 8
_D = 1024
_NTOK = _B * _T

_SC_CORES = 2
_SC_SUBCORES = 16
_NW = _SC_CORES * _SC_SUBCORES
_TPW = _NTOK // _NW
_CH = 8
_RPC = _CH * _ND
_NCHUNK = _TPW // _CH


def _make_sc_gather_sum():
    mesh = plsc.VectorSubcoreMesh(core_axis_name="c", subcore_axis_name="s")

    @functools.partial(
        pl.kernel,
        mesh=mesh,
        out_type=jax.ShapeDtypeStruct((_NTOK, _D), jnp.float32),
        scratch_types=[
            pltpu.VMEM((_TPW * _ND,), jnp.int32),
            pltpu.VMEM((_RPC, _D), jnp.float32),
            pltpu.VMEM((_RPC, _D), jnp.float32),
            pltpu.VMEM((_CH, _D), jnp.float32),
            pltpu.VMEM((_CH, _D), jnp.float32),
            pltpu.SemaphoreType.DMA,
            pltpu.SemaphoreType.DMA,
            pltpu.SemaphoreType.DMA,
            pltpu.SemaphoreType.DMA,
        ],
    )
    def gather_sum(table_hbm, idx_hbm, out_hbm, idx_v, buf_a, buf_b,
                   acc_a, acc_b, sem_a, sem_b, sem_oa, sem_ob):
        wid = lax.axis_index("s") * _SC_CORES + lax.axis_index("c")
        tok0 = wid * _TPW
        pltpu.sync_copy(idx_hbm.at[pl.ds(tok0 * _ND, _TPW * _ND)], idx_v)

        def compute(buf, acc):
            def cbody(i, carry):
                t = i >> 4
                dd = i & 15
                for u in range(4):
                    s = pl.ds(dd * 64 + u * 16, 16)
                    acc[t, s] = ((buf[4 * t + 0, s] + buf[4 * t + 1, s])
                                 + (buf[4 * t + 2, s] + buf[4 * t + 3, s]))
                return carry
            lax.fori_loop(0, _CH * 16, cbody, 0)

        def wait_gather(buf, sem):
            pltpu.make_async_copy(
                table_hbm.at[idx_v.at[pl.ds(0, _RPC)]], buf, sem).wait()

        def wait_out(acc, sem):
            pltpu.make_async_copy(
                acc, out_hbm.at[pl.ds(tok0, _CH)], sem).wait()

        pltpu.async_copy(table_hbm.at[idx_v.at[pl.ds(0, _RPC)]], buf_a, sem_a)

        def pbody(p, carry):
            c0 = 2 * p
            pltpu.async_copy(
                table_hbm.at[idx_v.at[pl.ds((c0 + 1) * _RPC, _RPC)]],
                buf_b, sem_b)
            wait_gather(buf_a, sem_a)

            @pl.when(p > 0)
            def _():
                wait_out(acc_a, sem_oa)
            compute(buf_a, acc_a)
            pltpu.async_copy(
                acc_a, out_hbm.at[pl.ds(tok0 + c0 * _CH, _CH)], sem_oa)

            @pl.when(p + 1 < _NCHUNK // 2)
            def _():
                pltpu.async_copy(
                    table_hbm.at[idx_v.at[pl.ds((c0 + 2) * _RPC, _RPC)]],
                    buf_a, sem_a)
            wait_gather(buf_b, sem_b)

            @pl.when(p > 0)
            def _():
                wait_out(acc_b, sem_ob)
            compute(buf_b, acc_b)
            pltpu.async_copy(
                acc_b, out_hbm.at[pl.ds(tok0 + (c0 + 1) * _CH, _CH)], sem_ob)
            return carry

        lax.fori_loop(0, _NCHUNK // 2, pbody, 0)
        wait_out(acc_a, sem_oa)
        wait_out(acc_b, sem_ob)

    return gather_sum


@functools.lru_cache(maxsize=1)
def _sc_gather_sum_cached():
    return _make_sc_gather_sum()

_TT = 64


def _expand_body(vec_ref, cont_ref, w_ref, comp_ref, lb_ref, out_ref):
    base = jnp.sum(comp_ref[...], axis=0) + jnp.sum(lb_ref[...], axis=0)[None, :]
    tok = vec_ref[...] + jnp.dot(cont_ref[...], w_ref[...],
                                 preferred_element_type=jnp.float32)
    out_ref[...] = tok[:, None, :] + base[None, :, :]


def _expand(vec, cont, w2d, comp, lin_b):
    return pl.pallas_call(
        _expand_body,
        grid=(_NTOK // _TT,),
        in_specs=[
            pl.BlockSpec((_TT, _D), lambda i: (i, 0)),
            pl.BlockSpec((_TT, _NC), lambda i: (i, 0)),
            pl.BlockSpec((_NC, _D), lambda i: (0, 0)),
            pl.BlockSpec((_ND + _NC, _SA, _D), lambda i: (0, 0, 0)),
            pl.BlockSpec((_NC, _D), lambda i: (0, 0)),
        ],
        out_specs=pl.BlockSpec((_TT, _SA, _D), lambda i: (i, 0, 0)),
        out_shape=jax.ShapeDtypeStruct((_NTOK, _SA, _D), jnp.float32),
        compiler_params=pltpu.CompilerParams(
            dimension_semantics=("arbitrary",)),
    )(vec, cont, w2d, comp, lin_b)


def kernel(discrete_actions, continuous_actions, emb_tables, lin_w, lin_b,
           component_tokens):
    table = emb_tables.reshape(_ND * _BINS, _D)
    idx = (discrete_actions.reshape(_NTOK, _ND).astype(jnp.int32)
           + (jnp.arange(_ND, dtype=jnp.int32) * _BINS)[None, :]).reshape(-1)
    vec = _sc_gather_sum_cached()(table, idx)
    cont = continuous_actions.reshape(_NTOK, _NC)
    w2d = lin_w[:, :, 0]
    comp = component_tokens.reshape(_ND + _NC, _SA, _D)
    out = _expand(vec, cont, w2d, comp, lin_b)
    return out.reshape(_B, _T, _SA, _D)


# speedup vs baseline: 2.3212x; 1.2385x over previous
import functools

import jax
import jax.numpy as jnp
from jax import lax
from jax.experimental import pallas as pl
from jax.experimental.pallas import tpu as pltpu
from jax.experimental.pallas import tpu_sc as plsc

_B = 16
_T = 256
_ND = 4
_NC = 6
_BINS = 256
_SA = 8
_D = 1024
_NTOK = _B * _T

_SC_CORES = 2
_SC_SUBCORES = 16
_NW = _SC_CORES * _SC_SUBCORES
_TPW = _NTOK // _NW
_CH = 8
_RPC = _CH * _ND
_NCHUNK = _TPW // _CH


def _make_sc_gather_sum():
    mesh = plsc.VectorSubcoreMesh(core_axis_name="c", subcore_axis_name="s")

    @functools.partial(
        pl.kernel,
        mesh=mesh,
        out_type=jax.ShapeDtypeStruct((_NTOK, _D), jnp.float32),
        scratch_types=[
            pltpu.VMEM((_TPW * _ND,), jnp.int32),
            pltpu.VMEM((_RPC, _D), jnp.float32),
            pltpu.VMEM((_RPC, _D), jnp.float32),
            pltpu.VMEM((_CH, _D), jnp.float32),
            pltpu.VMEM((_CH, _D), jnp.float32),
            pltpu.SemaphoreType.DMA,
            pltpu.SemaphoreType.DMA,
            pltpu.SemaphoreType.DMA,
            pltpu.SemaphoreType.DMA,
        ],
    )
    def gather_sum(table_hbm, idx_hbm, out_hbm, idx_v, buf_a, buf_b,
                   acc_a, acc_b, sem_a, sem_b, sem_oa, sem_ob):
        wid = lax.axis_index("s") * _SC_CORES + lax.axis_index("c")
        tok0 = wid * _TPW
        pltpu.sync_copy(idx_hbm.at[pl.ds(tok0 * _ND, _TPW * _ND)], idx_v)

        def compute(buf, acc):
            @plsc.parallel_loop(0, _CH * 16, 1, unroll=4)
            def cbody(i):
                t = i >> 4
                dd = i & 15
                for u in range(4):
                    sl = pl.ds(dd * 64 + u * 16, 16)
                    acc[t, sl] = ((buf[4 * t + 0, sl] + buf[4 * t + 1, sl])
                                  + (buf[4 * t + 2, sl] + buf[4 * t + 3, sl]))

        def wait_gather(buf, sem):
            pltpu.make_async_copy(
                table_hbm.at[idx_v.at[pl.ds(0, _RPC)]], buf, sem).wait()

        def wait_out(acc, sem):
            pltpu.make_async_copy(
                acc, out_hbm.at[pl.ds(tok0, _CH)], sem).wait()

        pltpu.async_copy(table_hbm.at[idx_v.at[pl.ds(0, _RPC)]], buf_a, sem_a)

        def pbody(p, carry):
            c0 = 2 * p
            pltpu.async_copy(
                table_hbm.at[idx_v.at[pl.ds((c0 + 1) * _RPC, _RPC)]],
                buf_b, sem_b)
            wait_gather(buf_a, sem_a)

            @pl.when(p > 0)
            def _():
                wait_out(acc_a, sem_oa)
            compute(buf_a, acc_a)
            pltpu.async_copy(
                acc_a, out_hbm.at[pl.ds(tok0 + c0 * _CH, _CH)], sem_oa)

            @pl.when(p + 1 < _NCHUNK // 2)
            def _():
                pltpu.async_copy(
                    table_hbm.at[idx_v.at[pl.ds((c0 + 2) * _RPC, _RPC)]],
                    buf_a, sem_a)
            wait_gather(buf_b, sem_b)

            @pl.when(p > 0)
            def _():
                wait_out(acc_b, sem_ob)
            compute(buf_b, acc_b)
            pltpu.async_copy(
                acc_b, out_hbm.at[pl.ds(tok0 + (c0 + 1) * _CH, _CH)], sem_ob)
            return carry

        lax.fori_loop(0, _NCHUNK // 2, pbody, 0)
        wait_out(acc_a, sem_oa)
        wait_out(acc_b, sem_ob)

    return gather_sum


@functools.lru_cache(maxsize=1)
def _sc_gather_sum_cached():
    return _make_sc_gather_sum()


_TT = 64


def _expand_body(vec_ref, cont_ref, w_ref, comp_ref, lb_ref, out_ref):
    base = jnp.sum(comp_ref[...], axis=0) + jnp.sum(lb_ref[...], axis=0)[None, :]
    tok = (vec_ref[...].astype(jnp.float32)
           + jnp.dot(cont_ref[...], w_ref[...],
                     preferred_element_type=jnp.float32))
    out_ref[...] = tok[:, None, :] + base[None, :, :]


def _expand(vec, cont, w2d, comp, lin_b):
    return pl.pallas_call(
        _expand_body,
        grid=(_NTOK // _TT,),
        in_specs=[
            pl.BlockSpec((_TT, _D), lambda i: (i, 0)),
            pl.BlockSpec((_TT, _NC), lambda i: (i, 0)),
            pl.BlockSpec((_NC, _D), lambda i: (0, 0)),
            pl.BlockSpec((_ND + _NC, _SA, _D), lambda i: (0, 0, 0)),
            pl.BlockSpec((_NC, _D), lambda i: (0, 0)),
        ],
        out_specs=pl.BlockSpec((_TT, _SA, _D), lambda i: (i, 0, 0)),
        out_shape=jax.ShapeDtypeStruct((_NTOK, _SA, _D), jnp.float32),
        compiler_params=pltpu.CompilerParams(
            dimension_semantics=("arbitrary",)),
    )(vec, cont, w2d, comp, lin_b)


def kernel(discrete_actions, continuous_actions, emb_tables, lin_w, lin_b,
           component_tokens):
    table = emb_tables.reshape(_ND * _BINS, _D)
    idx = (discrete_actions.reshape(_NTOK, _ND).astype(jnp.int32)
           + (jnp.arange(_ND, dtype=jnp.int32) * _BINS)[None, :]).reshape(-1)
    vec = _sc_gather_sum_cached()(table, idx)
    cont = continuous_actions.reshape(_NTOK, _NC)
    w2d = lin_w[:, :, 0]
    comp = component_tokens.reshape(_ND + _NC, _SA, _D)
    out = _expand(vec, cont, w2d, comp, lin_b)
    return out.reshape(_B, _T, _SA, _D)


# speedup vs baseline: 2.4063x; 1.0367x over previous
import functools

import jax
import jax.numpy as jnp
from jax import lax
from jax.experimental import pallas as pl
from jax.experimental.pallas import tpu as pltpu
from jax.experimental.pallas import tpu_sc as plsc

_B = 16
_T = 256
_ND = 4
_NC = 6
_BINS = 256
_SA = 8
_D = 1024
_NTOK = _B * _T

_NSLICE = 4
_SLICE = _NTOK // _NSLICE

_SC_CORES = 2
_SC_SUBCORES = 16
_NW = _SC_CORES * _SC_SUBCORES
_CH = 8
_RPC = _CH * _ND


def _make_sc_gather_sum(ntok):
    tpw = ntok // _NW
    nchunk = tpw // _CH
    mesh = plsc.VectorSubcoreMesh(core_axis_name="c", subcore_axis_name="s")

    @functools.partial(
        pl.kernel,
        mesh=mesh,
        out_type=jax.ShapeDtypeStruct((ntok, _D), jnp.float32),
        scratch_types=[
            pltpu.VMEM((tpw * _ND,), jnp.int32),
            pltpu.VMEM((_RPC, _D), jnp.float32),
            pltpu.VMEM((_RPC, _D), jnp.float32),
            pltpu.VMEM((_CH, _D), jnp.float32),
            pltpu.VMEM((_CH, _D), jnp.float32),
            pltpu.SemaphoreType.DMA,
            pltpu.SemaphoreType.DMA,
            pltpu.SemaphoreType.DMA,
            pltpu.SemaphoreType.DMA,
        ],
    )
    def gather_sum(table_hbm, idx_hbm, out_hbm, idx_v, buf_a, buf_b,
                   acc_a, acc_b, sem_a, sem_b, sem_oa, sem_ob):
        wid = lax.axis_index("s") * _SC_CORES + lax.axis_index("c")
        tok0 = wid * tpw
        pltpu.sync_copy(idx_hbm.at[pl.ds(tok0 * _ND, tpw * _ND)], idx_v)

        def compute(buf, acc):
            @plsc.parallel_loop(0, _CH * 16, 1, unroll=4)
            def cbody(i):
                t = i >> 4
                dd = i & 15
                for u in range(4):
                    sl = pl.ds(dd * 64 + u * 16, 16)
                    acc[t, sl] = ((buf[4 * t + 0, sl] + buf[4 * t + 1, sl])
                                  + (buf[4 * t + 2, sl] + buf[4 * t + 3, sl]))

        def wait_gather(buf, sem):
            pltpu.make_async_copy(
                table_hbm.at[idx_v.at[pl.ds(0, _RPC)]], buf, sem).wait()

        def wait_out(acc, sem):
            pltpu.make_async_copy(
                acc, out_hbm.at[pl.ds(tok0, _CH)], sem).wait()

        pltpu.async_copy(table_hbm.at[idx_v.at[pl.ds(0, _RPC)]], buf_a, sem_a)

        def pbody(p, carry):
            c0 = 2 * p
            pltpu.async_copy(
                table_hbm.at[idx_v.at[pl.ds((c0 + 1) * _RPC, _RPC)]],
                buf_b, sem_b)
            wait_gather(buf_a, sem_a)

            @pl.when(p > 0)
            def _():
                wait_out(acc_a, sem_oa)
            compute(buf_a, acc_a)
            pltpu.async_copy(
                acc_a, out_hbm.at[pl.ds(tok0 + c0 * _CH, _CH)], sem_oa)

            @pl.when(p + 1 < nchunk // 2)
            def _():
                pltpu.async_copy(
                    table_hbm.at[idx_v.at[pl.ds((c0 + 2) * _RPC, _RPC)]],
                    buf_a, sem_a)
            wait_gather(buf_b, sem_b)

            @pl.when(p > 0)
            def _():
                wait_out(acc_b, sem_ob)
            compute(buf_b, acc_b)
            pltpu.async_copy(
                acc_b, out_hbm.at[pl.ds(tok0 + (c0 + 1) * _CH, _CH)], sem_ob)
            return carry

        lax.fori_loop(0, nchunk // 2, pbody, 0)
        wait_out(acc_a, sem_oa)
        wait_out(acc_b, sem_ob)

    return gather_sum


@functools.lru_cache(maxsize=2)
def _sc_gather_sum_cached(ntok):
    return _make_sc_gather_sum(ntok)


_TT = 64


def _expand_first_body(vec_ref, cont_ref, w_ref, comp_ref, lb_ref, out_ref):
    base = jnp.sum(comp_ref[...], axis=0) + jnp.sum(lb_ref[...], axis=0)[None, :]
    tok = vec_ref[...] + jnp.dot(cont_ref[...], w_ref[...],
                                 preferred_element_type=jnp.float32)
    out_ref[...] = tok[:, None, :] + base[None, :, :]


def _expand_chain_body(prev_ref, vec_ref, cont_ref, w_ref, comp_ref, lb_ref,
                       out_ref):
    del prev_ref
    _expand_first_body(vec_ref, cont_ref, w_ref, comp_ref, lb_ref, out_ref)


def _expand_slice(k, prev, vec, cont, w2d, comp, lin_b):
    nblk = _SLICE // _TT
    data_specs = [
        pl.BlockSpec((_TT, _D), lambda i: (i, 0)),
        pl.BlockSpec((_TT, _NC), lambda i: (i, 0)),
        pl.BlockSpec((_NC, _D), lambda i: (0, 0)),
        pl.BlockSpec((_ND + _NC, _SA, _D), lambda i: (0, 0, 0)),
        pl.BlockSpec((_NC, _D), lambda i: (0, 0)),
    ]
    out_spec = pl.BlockSpec((_TT, _SA, _D),
                            lambda i, _k=k: (_k * nblk + i, 0, 0))
    out_shape = jax.ShapeDtypeStruct((_NTOK, _SA, _D), jnp.float32)
    params = pltpu.CompilerParams(dimension_semantics=("arbitrary",))
    if prev is None:
        return pl.pallas_call(
            _expand_first_body,
            grid=(nblk,),
            in_specs=data_specs,
            out_specs=out_spec,
            out_shape=out_shape,
            compiler_params=params,
        )(vec, cont, w2d, comp, lin_b)
    return pl.pallas_call(
        _expand_chain_body,
        grid=(nblk,),
        in_specs=[pl.BlockSpec(memory_space=pl.ANY)] + data_specs,
        out_specs=out_spec,
        out_shape=out_shape,
        input_output_aliases={0: 0},
        compiler_params=params,
    )(prev, vec, cont, w2d, comp, lin_b)


def kernel(discrete_actions, continuous_actions, emb_tables, lin_w, lin_b,
           component_tokens):
    table = emb_tables.reshape(_ND * _BINS, _D)
    idx = (discrete_actions.reshape(_NTOK, _ND).astype(jnp.int32)
           + (jnp.arange(_ND, dtype=jnp.int32) * _BINS)[None, :]).reshape(-1)
    cont = continuous_actions.reshape(_NTOK, _NC)
    w2d = lin_w[:, :, 0]
    comp = component_tokens.reshape(_ND + _NC, _SA, _D)

    sc = _sc_gather_sum_cached(_SLICE)
    vecs = [sc(table, idx[k * _SLICE * _ND:(k + 1) * _SLICE * _ND])
            for k in range(_NSLICE)]
    out = None
    for k in range(_NSLICE):
        out = _expand_slice(
            k, out, vecs[k],
            cont[k * _SLICE:(k + 1) * _SLICE], w2d, comp, lin_b)
    return out.reshape(_B, _T, _SA, _D)
